# baseline (speedup 1.0000x reference)
import numpy as np
import jax
import jax.numpy as jnp
from jax import lax
from jax.experimental import pallas as pl
from jax.experimental.pallas import tpu as pltpu

_SCALES = (8, 16, 32)
_KPS = 4
_KNN = 20
_NPI = 50
_B = 2
_HW = 32
_D = 768
_NG = len(_SCALES) * _B


def _pool_matrix(s):
    bs = _HW // s
    P = np.zeros((s * s, _HW * _HW), np.float32)
    for i in range(s):
        for j in range(s):
            for y in range(i * bs, (i + 1) * bs):
                for x in range(j * bs, (j + 1) * bs):
                    P[i * s + j, y * _HW + x] = 1.0 / (bs * bs)
    return P


def _rng_consts():
    key = jax.random.key(42)
    vinits, gumbs, c0s = [], [], []
    for s in _SCALES:
        n = s * s
        vinit = np.zeros((_B, _KPS - 1, n), np.float32)
        gumb = np.zeros((_B, _KPS - 1, n), np.float32)
        for b in range(_B):
            for i in range(_KPS - 1):
                key, sk = jax.random.split(key)
                vinit[b, i] = np.asarray(jax.random.normal(sk, (n,), jnp.float32))
            key, sk = jax.random.split(key)
            c0s.append(int(jax.random.randint(sk, (), 0, n)))
            for j in range(_KPS - 1):
                key, sk = jax.random.split(key)
                gumb[b, j] = np.asarray(jax.random.gumbel(sk, (n,), jnp.float32))
        vinits.append(vinit)
        gumbs.append(gumb)
    return vinits, gumbs, np.asarray(c0s, np.int32)


_POOL = {s: _pool_matrix(s) for s in _SCALES if s != _HW}
_VINITS, _GUMBS, _C0 = _rng_consts()

_F32 = jnp.float32
_DN_1_1 = (((1,), (1,)), ((), ()))
_DN_1_0 = (((1,), (0,)), ((), ()))
_DN_0_0 = (((0,), (0,)), ((), ()))


def _mm(a, b, dn):
    return lax.dot_general(a, b, dn, preferred_element_type=_F32)


def _mmh(a, b, dn):
    return lax.dot_general(a, b, dn, preferred_element_type=_F32,
                           precision=lax.Precision.HIGHEST)


def _mega_body(x_ref, p8_ref, p16_ref, v8_ref, v16_ref, v32_ref,
               g8_ref, g16_ref, g32_ref, c0_ref, out_ref):
    ones_d = jnp.ones((1, _D), _F32)
    p_refs = {8: p8_ref, 16: p16_ref}
    v_refs = {8: v8_ref, 16: v16_ref, 32: v32_ref}
    g_refs = {8: g8_ref, 16: g16_ref, 32: g32_ref}

    fs, fn_cols, dmats, cols = [], [], [], []
    for si, s in enumerate(_SCALES):
        n = s * s
        for b in range(_B):
            x = x_ref[b]
            f = x if s == _HW else _mm(p_refs[s][...], x, _DN_1_0)
            fsq = f * f
            fn_col = _mm(fsq, ones_d, _DN_1_1)
            fn_row = _mm(ones_d, fsq, _DN_1_1)
            G = _mm(f, f, _DN_1_1)
            dmat = jnp.sqrt(jnp.maximum(fn_col + fn_row - 2.0 * G, 0.0))
            fs.append(f)
            fn_cols.append(fn_col)
            dmats.append(dmat)
            cols.append(lax.broadcasted_iota(jnp.int32, (n, n), 1).astype(_F32))

    big = _F32(np.inf)

    def _extract(alive_d, alive_i):
        md = jnp.min(alive_d, axis=1, keepdims=True)
        mi = jnp.min(jnp.where(alive_d == md, alive_i, big),
                     axis=1, keepdims=True)
        return md, mi

    def thr_body(_, carry):
        out = []
        for (md, mi), d, c in zip(carry, dmats, cols):
            dead = (d < md) | ((d == md) & (c <= mi))
            out.append(_extract(jnp.where(dead, big, d),
                                jnp.where(dead, big, c)))
        return tuple(out)

    thrs = lax.fori_loop(
        0, _KNN, thr_body,
        tuple(_extract(d, c) for d, c in zip(dmats, cols)))

    Ms = []
    for g in range(_NG):
        n = dmats[g].shape[0]
        d, c = dmats[g], cols[g]
        td, ti = thrs[g]
        ri = lax.broadcasted_iota(jnp.int32, (n, n), 0).astype(_F32)
        sel = ((d < td) | ((d == td) & (c <= ti))) & (c != ri)
        A = jnp.where(sel, jnp.exp(d * -0.5), 0.0)
        eye_f = jnp.where(c == ri, _F32(1.0), _F32(0.0))
        At = _mm(A, eye_f, _DN_0_0)
        Wm = 0.5 * (A + At)
        ones_n = jnp.ones((1, n), _F32)
        Dv = _mm(ones_n, Wm, _DN_1_1)
        Dis = lax.rsqrt(Dv + 1e-8)
        Dis_col = _mm(Dis, eye_f, _DN_1_1)
        Ms.append(Wm * Dis * Dis_col)

    def norm1(v):
        return v / (jnp.sqrt(jnp.sum(v * v)) + 1e-8)

    evs = [[] for _ in range(_NG)]
    for i in range(_KPS):
        v0s, M2s = [], []
        for g in range(_NG):
            si, b = divmod(g, _B)
            if i == 0:
                v0 = _mm(ones_d, fs[g], _DN_1_1) * _F32(1.0 / _D)
                Mt = Ms[g]
            else:
                v0 = v_refs[_SCALES[si]][b, i - 1:i, :]
                E = jnp.concatenate(evs[g], axis=0)
                EM = _mmh(E, Ms[g], _DN_1_0)
                Mt = Ms[g] - _mmh(E, EM, _DN_0_0)
            v0s.append(norm1(v0))
            M2s.append(_mm(Mt, Mt, _DN_1_0))

        def pbody(_, vs, M2s_t=tuple(M2s)):
            out = []
            for g in range(_NG):
                vn = vs[g]
                for _u in range(5):
                    vn = _mm(vn, M2s_t[g], _DN_1_1)
                out.append(norm1(vn))
            return tuple(out)

        vs = lax.fori_loop(0, _NPI // 10, pbody, tuple(v0s))
        for g in range(_NG):
            evs[g].append(vs[g])

    fnorms, Es, ils, c_idx, min_d = [], [], [], [], [None] * _NG
    for g in range(_NG):
        n = fs[g].shape[0]
        fnorms.append(fs[g] / jnp.maximum(jnp.sqrt(fn_cols[g]), 1e-12))
        Es.append(jnp.concatenate(evs[g], axis=0))
        ils.append(lax.broadcasted_iota(jnp.int32, (1, n), 1))
        c_idx.append(c0_ref[g])

    for j in range(_KPS):
        for g in range(_NG):
            si, b = divmod(g, _B)
            n = fs[g].shape[0]
            il = ils[g]
            oh = jnp.where(il == c_idx[g], _F32(1.0), _F32(0.0))
            row = si * _KPS + j
            out_ref[b, row:row + 1, :] = _mmh(oh, fs[g], _DN_1_0)
            if j == _KPS - 1:
                continue
            fcn = _mmh(oh, fnorms[g], _DN_1_0)
            diff = fnorms[g] - fcn
            d2fn = _mm(ones_d, diff * diff, _DN_1_1)
            Ec = _mmh(Es[g], oh, _DN_1_1)
            dE = 2.0 * Es[g] - 2.0 * Ec
            d2E = jnp.sum(dE * dE, axis=0, keepdims=True)
            dcur = jnp.sqrt(jnp.maximum(d2E + d2fn, 0.0))
            min_d[g] = dcur if min_d[g] is None else jnp.minimum(min_d[g], dcur)
            probs = min_d[g] * min_d[g]
            probs = probs / (jnp.sum(probs) + 1e-8)
            score = jnp.log(probs + 1e-20) + g_refs[_SCALES[si]][b, j:j + 1, :]
            mx = jnp.max(score)
            c_idx[g] = jnp.min(jnp.where(score == mx, il, n))


def kernel(features):
    X = features.reshape(_B, _HW * _HW, _D)
    full = lambda a: pl.BlockSpec(a.shape, lambda: tuple(0 for _ in a.shape))
    args = (X, _POOL[8], _POOL[16], _VINITS[0], _VINITS[1], _VINITS[2],
            _GUMBS[0], _GUMBS[1], _GUMBS[2])
    return pl.pallas_call(
        _mega_body,
        in_specs=[full(a) for a in args] + [pl.BlockSpec(memory_space=pltpu.SMEM)],
        out_specs=pl.BlockSpec((_B, len(_SCALES) * _KPS, _D), lambda: (0, 0, 0)),
        out_shape=jax.ShapeDtypeStruct((_B, len(_SCALES) * _KPS, _D), _F32),
    )(*args, _C0)
